# baseline (speedup 1.0000x reference)
import jax
import jax.numpy as jnp
from jax.experimental import pallas as pl
from jax.experimental.pallas import tpu as pltpu
from jax.experimental.pallas import tpu_sc as plsc

_B = 4096
_S = 200
_H = 64
_BS = 10
_NSUB = 32
_CW = _B // _NSUB
_NK = _CW // 16


def _sc_positions(ids_hbm, pos_hbm, ids_v, pos_v, sem):
    wid = jax.lax.axis_index("s") * 2 + jax.lax.axis_index("c")
    base = wid * _CW
    pltpu.async_copy(ids_hbm.at[pl.ds(base, _CW), :], ids_v, sem).wait()
    lane = jax.lax.iota(jnp.int32, 16)

    def row_body(r, posvec):
        r16 = jnp.zeros((16,), jnp.int32) + r
        cnt = jnp.zeros((16,), jnp.int32)
        for i in range(_S // 16):
            v = plsc.load_gather(ids_v, [r16, lane + (i * 16)])
            cnt += (v != 0).astype(jnp.int32)
        tail = plsc.load_gather(ids_v, [r16, lane + (_S - 16)])
        tail_new = 16 - (_S % 16)
        cnt += ((lane >= tail_new) & (tail != 0)).astype(jnp.int32)
        total = jnp.sum(cnt)
        pos = jnp.clip(total - 1, 0, _S - 1)
        idv16 = plsc.load_gather(ids_v, [r16, jnp.zeros((16,), jnp.int32) + pos])
        idv = jnp.sum(jnp.where(lane == 0, idv16, 0))
        posv = jnp.where(idv != 0, pos, -1)
        return jnp.where(lane == jax.lax.rem(r, 16), posv, posvec)

    def grp_body(g, carry):
        posvec = jax.lax.fori_loop(g * 16, g * 16 + 16, row_body,
                                   jnp.zeros((16,), jnp.int32))
        pos_v[pl.ds(g * 16, 16)] = posvec
        return carry

    jax.lax.fori_loop(0, _CW // 16, grp_body, 0)
    pltpu.sync_copy(pos_v, pos_hbm.at[pl.ds(base, _CW)])


def _tc_kernel(pos_ref, emb_ref, x_ref, o_ref):
    step = pl.program_id(0)
    pos = pos_ref[...].reshape(1, 1, _B)
    x = x_ref[...]
    srel = pos - step * _BS
    sl_iota = jax.lax.broadcasted_iota(jnp.int32, (_BS, 1, _B), 0)
    m = sl_iota == srel
    o_ref[...] = jnp.where(m, emb_ref[...], x)


def kernel(inputs, item_ids, masked_item_embedding):
    x_t = jnp.transpose(inputs, (1, 2, 0))
    emb3 = masked_item_embedding.reshape(1, _H, 1)

    sc_call = pl.kernel(
        _sc_positions,
        out_type=jax.ShapeDtypeStruct((_B,), jnp.int32),
        mesh=plsc.VectorSubcoreMesh(core_axis_name="c", subcore_axis_name="s"),
        scratch_types=[
            pltpu.VMEM((_CW, _S), jnp.int32),
            pltpu.VMEM((_CW,), jnp.int32),
            pltpu.SemaphoreType.DMA,
        ],
        compiler_params=pltpu.CompilerParams(needs_layout_passes=False),
    )
    pos = sc_call(item_ids)
    pos2 = pos.reshape(1, _B)

    out_t = pl.pallas_call(
        _tc_kernel,
        grid=(_S // _BS,),
        in_specs=[
            pl.BlockSpec((1, _B), lambda i: (0, 0)),
            pl.BlockSpec((1, _H, 1), lambda i: (0, 0, 0)),
            pl.BlockSpec((_BS, _H, _B), lambda i: (i, 0, 0)),
        ],
        out_specs=pl.BlockSpec((_BS, _H, _B), lambda i: (i, 0, 0)),
        out_shape=jax.ShapeDtypeStruct((_S, _H, _B), inputs.dtype),
    )(pos2, emb3, x_t)
    return jnp.transpose(out_t, (2, 0, 1))


# speedup vs baseline: 1.1822x; 1.1822x over previous
import jax
import jax.numpy as jnp
from jax.experimental import pallas as pl
from jax.experimental.pallas import tpu as pltpu

_B = 4096
_S = 200
_H = 64
_BS = 10


def _plm_kernel(ids_ref, emb_ref, x_ref, o_ref, pos_ref):
    step = pl.program_id(0)

    @pl.when(step == 0)
    def _():
        ids = ids_ref[...]
        nz = (ids != 0).astype(jnp.int32)
        cnt = jnp.sum(nz, axis=0, keepdims=True)
        pos = jnp.clip(cnt - 1, 0, _S - 1)
        s_iota = jax.lax.broadcasted_iota(jnp.int32, ids.shape, 0)
        idv = jnp.sum(jnp.where(s_iota == pos, ids, 0), axis=0, keepdims=True)
        pos_ref[...] = jnp.where(idv != 0, pos, -1)

    pos = pos_ref[...]
    x = x_ref[...]
    srel = (pos - step * _BS).reshape(1, 1, _B)
    sl_iota = jax.lax.broadcasted_iota(jnp.int32, (_BS, 1, _B), 0)
    m = sl_iota == srel
    o_ref[...] = jnp.where(m, emb_ref[...], x)


def kernel(inputs, item_ids, masked_item_embedding):
    x_t = jnp.transpose(inputs, (1, 2, 0))
    ids_t = jnp.transpose(item_ids, (1, 0))
    emb3 = masked_item_embedding.reshape(1, _H, 1)
    out_t = pl.pallas_call(
        _plm_kernel,
        grid=(_S // _BS,),
        in_specs=[
            pl.BlockSpec((_S, _B), lambda i: (0, 0)),
            pl.BlockSpec((1, _H, 1), lambda i: (0, 0, 0)),
            pl.BlockSpec((_BS, _H, _B), lambda i: (i, 0, 0)),
        ],
        out_specs=pl.BlockSpec((_BS, _H, _B), lambda i: (i, 0, 0)),
        out_shape=jax.ShapeDtypeStruct((_S, _H, _B), inputs.dtype),
        scratch_shapes=[pltpu.VMEM((1, _B), jnp.int32)],
    )(ids_t, emb3, x_t)
    return jnp.transpose(out_t, (2, 0, 1))
